# Initial kernel scaffold; baseline (speedup 1.0000x reference)
import jax
import jax.numpy as jnp
from jax.experimental import pallas as pl


def kernel(x, edge_index, batch, W1, b1, W2, b2, Wp, bp):
    raise NotImplementedError("write your pallas kernel here")


# speedup vs baseline: 14.0460x; 14.0460x over previous
import dataclasses
import functools

import jax
import jax.numpy as jnp
from jax import lax
from jax.experimental import pallas as pl
from jax.experimental.pallas import tpu as pltpu
from jax.experimental.pallas import tpu_sc as plsc

N = 10000
D = 128
E = 320000
NC, NS = 2, 16
NT = NC * NS
CHUNK = 128
K = -(-E // (NT * CHUNK))
E_PAD = NT * K * CHUNK
N_ROWS = 10112
RPT = N_ROWS // NS

_mesh = plsc.VectorSubcoreMesh(core_axis_name="core", subcore_axis_name="subcore")

_sc_params = pltpu.CompilerParams()
if "needs_layout_passes" in pltpu.CompilerParams.__dataclass_fields__:
    _sc_params = dataclasses.replace(_sc_params, needs_layout_passes=False)


def _zero_rows(buf, nrows):
    z = jnp.zeros((16,), jnp.float32)

    @pl.loop(0, nrows)
    def _(r):
        for c in range(D // 16):
            buf[r, pl.ds(c * 16, 16)] = z


@functools.partial(
    pl.kernel,
    out_type=jax.ShapeDtypeStruct((NT, N_ROWS), jnp.float32),
    mesh=_mesh,
    compiler_params=_sc_params,
    scratch_types=[
        pltpu.VMEM((K, CHUNK), jnp.int32),
        pltpu.VMEM((N_ROWS,), jnp.float32),
    ],
)
def _sc_degree(dst_hbm, deg_hbm, idx_v, hist_v):
    c = lax.axis_index("core")
    s = lax.axis_index("subcore")
    t = c * NS + s

    z = jnp.zeros((16,), jnp.float32)

    @pl.loop(0, N_ROWS // 16)
    def _(i):
        hist_v[pl.ds(i * 16, 16)] = z

    pltpu.sync_copy(dst_hbm.at[t], idx_v)

    ones = jnp.ones((16,), jnp.float32)

    @pl.loop(0, K)
    def _(k):
        for cc in range(CHUNK // 16):
            v = idx_v[k, pl.ds(cc * 16, 16)]
            plsc.addupdate_scatter(hist_v, [v], ones)

    pltpu.sync_copy(hist_v, deg_hbm.at[t])


@functools.partial(
    pl.kernel,
    out_type=jax.ShapeDtypeStruct((NC, N_ROWS, D), jnp.float32),
    mesh=_mesh,
    compiler_params=_sc_params,
    scratch_types=[
        pltpu.VMEM((K, CHUNK), jnp.int32),
        pltpu.VMEM((K, CHUNK), jnp.int32),
        pltpu.VMEM((CHUNK, D), jnp.float32),
        pltpu.VMEM_SHARED((N_ROWS, D), jnp.float32),
    ],
)
def _sc_messages(y_hbm, src_hbm, dst_hbm, acc_hbm, src_v, dst_v, gbuf, acc_sh):
    c = lax.axis_index("core")
    s = lax.axis_index("subcore")
    t = c * NS + s

    _zero_rows(gbuf, CHUNK)
    base = s * RPT

    @pl.loop(0, RPT // CHUNK)
    def _(i):
        pltpu.sync_copy(gbuf, acc_sh.at[pl.ds(base + i * CHUNK, CHUNK)])

    rem = RPT % CHUNK
    if rem:
        pltpu.sync_copy(gbuf.at[pl.ds(0, rem)],
                        acc_sh.at[pl.ds(base + (RPT // CHUNK) * CHUNK, rem)])

    pltpu.sync_copy(src_hbm.at[t], src_v)
    pltpu.sync_copy(dst_hbm.at[t], dst_v)
    plsc.subcore_barrier()

    @pl.loop(0, K)
    def _(k):
        pltpu.sync_copy(y_hbm.at[src_v.at[k]], gbuf)
        pltpu.sync_copy(gbuf, acc_sh.at[dst_v.at[k]], add=True)

    plsc.subcore_barrier()

    @pl.loop(0, RPT // CHUNK)
    def _(i):
        pltpu.sync_copy(acc_sh.at[pl.ds(base + i * CHUNK, CHUNK)], gbuf)
        pltpu.sync_copy(gbuf, acc_hbm.at[c].at[pl.ds(base + i * CHUNK, CHUNK)])

    if rem:
        off = base + (RPT // CHUNK) * CHUNK
        pltpu.sync_copy(acc_sh.at[pl.ds(off, rem)], gbuf.at[pl.ds(0, rem)])
        pltpu.sync_copy(gbuf.at[pl.ds(0, rem)], acc_hbm.at[c].at[pl.ds(off, rem)])


def _mm_body(x_ref, w_ref, o_ref):
    o_ref[...] = jnp.dot(x_ref[...], w_ref[...],
                         preferred_element_type=jnp.float32,
                         precision=lax.Precision.HIGHEST)


def _tc_matmul(x, w):
    return pl.pallas_call(
        _mm_body,
        out_shape=jax.ShapeDtypeStruct((x.shape[0], w.shape[1]), jnp.float32),
    )(x, w)


def _scale_body(xw_ref, deg_ref, y_ref):
    dinv = lax.rsqrt(deg_ref[...])
    y_ref[:N, :] = xw_ref[...] * dinv[:N]
    y_ref[N:, :] = jnp.zeros((N_ROWS - N, D), jnp.float32)


def _tc_scale(xw, deg):
    return pl.pallas_call(
        _scale_body,
        out_shape=jax.ShapeDtypeStruct((N_ROWS, D), jnp.float32),
    )(xw, deg)


def _mid_body(acc_ref, y_ref, deg_ref, b_ref, w_ref, y2_ref):
    dinv = lax.rsqrt(deg_ref[...])
    pre = dinv * (acc_ref[0] + acc_ref[1] + y_ref[...]) + b_ref[...]
    h = jnp.maximum(pre, 0.0)
    y2 = jnp.dot(h, w_ref[...], preferred_element_type=jnp.float32,
                 precision=lax.Precision.HIGHEST) * dinv
    rows = lax.broadcasted_iota(jnp.int32, (N_ROWS, D), 0)
    y2_ref[...] = jnp.where(rows < N, y2, 0.0)


def _tc_mid(acc, y, deg, b, w):
    return pl.pallas_call(
        _mid_body,
        out_shape=jax.ShapeDtypeStruct((N_ROWS, D), jnp.float32),
    )(acc, y, deg, b, w)


def _out_body(acc_ref, y_ref, deg_ref, b_ref, xwp_ref, bp_ref, o_ref):
    dinv = lax.rsqrt(deg_ref[...])[:N]
    pre = dinv * (acc_ref[0, :N] + acc_ref[1, :N] + y_ref[:N, :]) + b_ref[...]
    h = jnp.maximum(pre, 0.0)
    o_ref[...] = h + xwp_ref[...] + bp_ref[...]


def _tc_out(acc, y, deg, b, xwp, bp):
    return pl.pallas_call(
        _out_body,
        out_shape=jax.ShapeDtypeStruct((N, D), jnp.float32),
    )(acc, y, deg, b, xwp, bp)


def kernel(x, edge_index, batch, W1, b1, W2, b2, Wp, bp):
    del batch
    src = edge_index[0].astype(jnp.int32)
    dst = edge_index[1].astype(jnp.int32)
    pad = E_PAD - E
    src = jnp.concatenate([src, jnp.full((pad,), N, jnp.int32)]).reshape(NT, K, CHUNK)
    dst = jnp.concatenate([dst, jnp.full((pad,), N, jnp.int32)]).reshape(NT, K, CHUNK)

    degp = _sc_degree(dst)
    xw1 = _tc_matmul(x, W1)
    xwp = _tc_matmul(x, Wp)
    deg = (degp.sum(axis=0) + 1.0).reshape(N_ROWS, 1)

    y1 = _tc_scale(xw1, deg)
    acc1 = _sc_messages(y1, src, dst)
    y2 = _tc_mid(acc1, y1, deg, b1.reshape(1, D), W2)
    acc2 = _sc_messages(y2, src, dst)
    return _tc_out(acc2, y2, deg, b2.reshape(1, D), xwp, bp.reshape(1, D))
